# baseline (speedup 1.0000x reference)
import functools

import jax
import jax.numpy as jnp
from jax import lax
from jax.experimental import pallas as pl
from jax.experimental.pallas import tpu as pltpu
from jax.experimental.pallas import tpu_sc as plsc

V = 1000000
D = 64
C = 128
B = 4096
H = 200
G = 3 * H
NC, NS, L = 2, 16, 16
NW = NC * NS
BPW = B // NW
GRP = 16
NGRP = BPW // GRP
CHUNKS = ((0, 104), (104, 96))


def _sc_pool(x, embed):
    mesh = plsc.VectorSubcoreMesh(core_axis_name="c", subcore_axis_name="s")

    @functools.partial(
        pl.kernel,
        mesh=mesh,
        compiler_params=pltpu.CompilerParams(use_tc_tiling_on_sc=False),
        out_type=jax.ShapeDtypeStruct((B, D), jnp.float32),
        scratch_types=[
            pltpu.VMEM((2, 3, GRP, H), jnp.int32),
            pltpu.VMEM((2, G, D), jnp.float32),
            pltpu.VMEM((BPW, D), jnp.float32),
            pltpu.SemaphoreType.DMA,
            pltpu.SemaphoreType.DMA,
            pltpu.SemaphoreType.DMA,
        ],
    )
    def k(x_hbm, emb_hbm, out_hbm, idx_v, rows_v, acc_v, sem_i, sem_r0, sem_r1):
        wid = lax.axis_index("s") * NC + lax.axis_index("c")
        base = wid * BPW

        def issue_idx(grp, buf):
            for g in range(3):
                pltpu.async_copy(
                    x_hbm.at[g, pl.ds(base + grp * GRP, GRP), :],
                    idx_v.at[buf, g],
                    sem_i,
                )

        def wait_idx():
            for g in range(3):
                pltpu.make_async_copy(
                    x_hbm.at[g, pl.ds(base, GRP), :], idx_v.at[0, g], sem_i
                ).wait()

        def issue_gathers(buf, r, rbuf):
            sem = sem_r0 if rbuf == 0 else sem_r1
            for g in range(3):
                for off, sz in CHUNKS:
                    pltpu.async_copy(
                        emb_hbm.at[idx_v.at[buf, g, r, pl.ds(off, sz)]],
                        rows_v.at[rbuf, pl.ds(g * H + off, sz)],
                        sem,
                    )

        def wait_gathers(rbuf):
            sem = sem_r0 if rbuf == 0 else sem_r1
            for g in range(3):
                for off, sz in CHUNKS:
                    pltpu.make_async_copy(
                        emb_hbm.at[pl.ds(0, sz)],
                        rows_v.at[rbuf, pl.ds(g * H + off, sz)],
                        sem,
                    ).wait()

        def accumulate(rbuf, out_row):
            def body(j, accs):
                return tuple(
                    a + rows_v[rbuf, j, pl.ds(k2 * L, L)]
                    for k2, a in enumerate(accs)
                )

            accs = lax.fori_loop(
                0, G, body,
                tuple(jnp.zeros((L,), jnp.float32) for _ in range(D // L)),
                unroll=8,
            )
            for k2, a in enumerate(accs):
                acc_v[out_row, pl.ds(k2 * L, L)] = a

        issue_idx(0, 0)
        wait_idx()
        issue_idx(1, 1)
        issue_gathers(0, 0, 0)

        def grp_body(grp, carry):
            buf = lax.rem(grp, 2)
            nbuf = lax.rem(grp + 1, 2)
            for r in range(GRP):
                rbuf = r % 2
                if r < GRP - 1:
                    issue_gathers(buf, r + 1, (r + 1) % 2)
                else:
                    wait_idx()

                    @pl.when(grp + 2 < NGRP)
                    def _():
                        issue_idx(grp + 2, buf)

                    issue_gathers(nbuf, 0, 0)
                wait_gathers(rbuf)
                accumulate(rbuf, grp * GRP + r)
            return carry

        lax.fori_loop(0, NGRP - 1, grp_body, 0)

        fbuf = (NGRP - 1) % 2
        for r in range(GRP):
            rbuf = r % 2
            if r < GRP - 1:
                issue_gathers(fbuf, r + 1, (r + 1) % 2)
            wait_gathers(rbuf)
            accumulate(rbuf, (NGRP - 1) * GRP + r)

        pltpu.sync_copy(acc_v, out_hbm.at[pl.ds(base, BPW)])

    return k(x, embed)


def _tc_logit(pooled, fc_w, fc_b2):
    BLK = 512

    def body(p_ref, w_ref, b_ref, o_ref):
        p = p_ref[...] * (1.0 / H)
        o_ref[...] = (
            lax.dot_general(
                p, w_ref[...], (((1,), (1,)), ((), ())),
                preferred_element_type=jnp.float32,
            )
            + b_ref[...]
        )

    return pl.pallas_call(
        body,
        grid=(B // BLK,),
        in_specs=[
            pl.BlockSpec((BLK, D), lambda i: (i, 0)),
            pl.BlockSpec((C, D), lambda i: (0, 0)),
            pl.BlockSpec((1, C), lambda i: (0, 0)),
        ],
        out_specs=pl.BlockSpec((BLK, C), lambda i: (i, 0)),
        out_shape=jax.ShapeDtypeStruct((B, C), jnp.float32),
    )(pooled, fc_w, fc_b2)


CB = 512
NB = (V + 2 * CB - 1) // (2 * CB)
RP = NB * CB


def _tc_compact(embed):
    embT = embed.T

    def body(a_ref, b_ref, o_ref):
        o_ref[...] = jnp.concatenate([a_ref[...].T, b_ref[...].T], axis=1)

    return pl.pallas_call(
        body,
        grid=(NB,),
        in_specs=[
            pl.BlockSpec((D, CB), lambda b: (0, 2 * b)),
            pl.BlockSpec((D, CB), lambda b: (0, 2 * b + 1)),
        ],
        out_specs=pl.BlockSpec((CB, 2 * D), lambda b: (b, 0)),
        out_shape=jax.ShapeDtypeStruct((RP, 2 * D), jnp.float32),
    )(embT, embT)


def _tc_remap_idx(x):
    XB = 512

    def body(x_ref, o_ref):
        v = x_ref[...]
        j = jnp.bitwise_and(v, 2 * CB - 1)
        o_ref[...] = v + j - jnp.where(j >= CB, 2 * CB - 1, 0)

    return pl.pallas_call(
        body,
        grid=(B // XB,),
        in_specs=[pl.BlockSpec((3, XB, H), lambda i: (0, i, 0))],
        out_specs=pl.BlockSpec((3, XB, H), lambda i: (0, i, 0)),
        out_shape=jax.ShapeDtypeStruct((3, B, H), jnp.int32),
    )(x)


def kernel(x, embed, embed_bigram, embed_trigram, fc_w, fc_b):
    tbl = _tc_compact(embed).reshape(2 * RP, D)
    pooled = _sc_pool(_tc_remap_idx(x), tbl)
    return _tc_logit(pooled, fc_w, fc_b.reshape(1, C))


# speedup vs baseline: 1.5774x; 1.5774x over previous
import functools

import jax
import jax.numpy as jnp
from jax import lax
from jax.experimental import pallas as pl
from jax.experimental.pallas import tpu as pltpu
from jax.experimental.pallas import tpu_sc as plsc

V = 1000000
D = 64
C = 128
B = 4096
H = 200
G = 3 * H
NC, NS, L = 2, 16, 16
NW = NC * NS
BPW = B // NW
GRP = 16
NGRP = BPW // GRP
CHUNKS = ((0, 104), (104, 96))


def _sc_pool(x, embed):
    mesh = plsc.VectorSubcoreMesh(core_axis_name="c", subcore_axis_name="s")

    @functools.partial(
        pl.kernel,
        mesh=mesh,
        compiler_params=pltpu.CompilerParams(use_tc_tiling_on_sc=False),
        out_type=jax.ShapeDtypeStruct((B, D), jnp.float32),
        scratch_types=[
            pltpu.VMEM((2, 3, GRP, H), jnp.int32),
            pltpu.VMEM((2, G, D), jnp.float32),
            pltpu.VMEM((BPW, D), jnp.float32),
            pltpu.SemaphoreType.DMA,
            pltpu.SemaphoreType.DMA,
            pltpu.SemaphoreType.DMA,
        ],
    )
    def k(x_hbm, emb_hbm, out_hbm, idx_v, rows_v, acc_v, sem_i, sem_r0, sem_r1):
        wid = lax.axis_index("s") * NC + lax.axis_index("c")
        base = wid * BPW

        def issue_idx(grp, buf):
            for g in range(3):
                pltpu.async_copy(
                    x_hbm.at[g, pl.ds(base + grp * GRP, GRP), :],
                    idx_v.at[buf, g],
                    sem_i,
                )

        def wait_idx():
            for g in range(3):
                pltpu.make_async_copy(
                    x_hbm.at[g, pl.ds(base, GRP), :], idx_v.at[0, g], sem_i
                ).wait()

        def issue_gathers(buf, r, rbuf):
            sem = sem_r0 if rbuf == 0 else sem_r1
            for g in range(3):
                for off, sz in CHUNKS:
                    pltpu.async_copy(
                        emb_hbm.at[idx_v.at[buf, g, r, pl.ds(off, sz)]],
                        rows_v.at[rbuf, pl.ds(g * H + off, sz)],
                        sem,
                    )

        def wait_gathers(rbuf):
            sem = sem_r0 if rbuf == 0 else sem_r1
            for g in range(3):
                for off, sz in CHUNKS:
                    pltpu.make_async_copy(
                        emb_hbm.at[pl.ds(0, sz)],
                        rows_v.at[rbuf, pl.ds(g * H + off, sz)],
                        sem,
                    ).wait()

        def accumulate(rbuf, out_row):
            def body(j, accs):
                return tuple(
                    a + rows_v[rbuf, j, pl.ds(k2 * L, L)]
                    for k2, a in enumerate(accs)
                )

            accs = lax.fori_loop(
                0, G, body,
                tuple(jnp.zeros((L,), jnp.float32) for _ in range(D // L)),
                unroll=8,
            )
            for k2, a in enumerate(accs):
                acc_v[out_row, pl.ds(k2 * L, L)] = a

        issue_idx(0, 0)
        wait_idx()
        issue_idx(1, 1)
        issue_gathers(0, 0, 0)

        def grp_body(grp, carry):
            buf = lax.rem(grp, 2)
            nbuf = lax.rem(grp + 1, 2)
            for r in range(GRP):
                rbuf = r % 2
                if r < GRP - 1:
                    issue_gathers(buf, r + 1, (r + 1) % 2)
                else:
                    wait_idx()

                    @pl.when(grp + 2 < NGRP)
                    def _():
                        issue_idx(grp + 2, buf)

                    issue_gathers(nbuf, 0, 0)
                wait_gathers(rbuf)
                accumulate(rbuf, grp * GRP + r)
            return carry

        lax.fori_loop(0, NGRP - 1, grp_body, 0)

        fbuf = (NGRP - 1) % 2
        for r in range(GRP):
            rbuf = r % 2
            if r < GRP - 1:
                issue_gathers(fbuf, r + 1, (r + 1) % 2)
            wait_gathers(rbuf)
            accumulate(rbuf, (NGRP - 1) * GRP + r)

        pltpu.sync_copy(acc_v, out_hbm.at[pl.ds(base, BPW)])

    return k(x, embed)


def _tc_logit(pooled, fc_w, fc_b2):
    BLK = 512

    def body(p_ref, w_ref, b_ref, o_ref):
        p = p_ref[...] * (1.0 / H)
        o_ref[...] = (
            lax.dot_general(
                p, w_ref[...], (((1,), (1,)), ((), ())),
                preferred_element_type=jnp.float32,
            )
            + b_ref[...]
        )

    return pl.pallas_call(
        body,
        grid=(B // BLK,),
        in_specs=[
            pl.BlockSpec((BLK, D), lambda i: (i, 0)),
            pl.BlockSpec((C, D), lambda i: (0, 0)),
            pl.BlockSpec((1, C), lambda i: (0, 0)),
        ],
        out_specs=pl.BlockSpec((BLK, C), lambda i: (i, 0)),
        out_shape=jax.ShapeDtypeStruct((B, C), jnp.float32),
    )(pooled, fc_w, fc_b2)


CB = 2048
NB = (V + 2 * CB - 1) // (2 * CB)
RP = NB * CB


def _tc_compact(embed):
    embT = embed.T

    def body(a_ref, b_ref, o_ref):
        eye = jnp.eye(D, dtype=jnp.float32)
        t = lambda m: lax.dot_general(
            m, eye, (((0,), (0,)), ((), ())), preferred_element_type=jnp.float32
        )
        o_ref[...] = jnp.concatenate([t(a_ref[...]), t(b_ref[...])], axis=1)

    return pl.pallas_call(
        body,
        grid=(NB,),
        in_specs=[
            pl.BlockSpec((D, CB), lambda b: (0, 2 * b)),
            pl.BlockSpec((D, CB), lambda b: (0, jnp.minimum(2 * b + 1, V // CB))),
        ],
        out_specs=pl.BlockSpec((CB, 2 * D), lambda b: (b, 0)),
        out_shape=jax.ShapeDtypeStruct((RP, 2 * D), jnp.float32),
    )(embT, embT)


def _tc_remap_idx(x):
    XB = 512

    def body(x_ref, o_ref):
        v = x_ref[...]
        j = jnp.bitwise_and(v, 2 * CB - 1)
        o_ref[...] = v + j - jnp.where(j >= CB, 2 * CB - 1, 0)

    return pl.pallas_call(
        body,
        grid=(B // XB,),
        in_specs=[pl.BlockSpec((3, XB, H), lambda i: (0, i, 0))],
        out_specs=pl.BlockSpec((3, XB, H), lambda i: (0, i, 0)),
        out_shape=jax.ShapeDtypeStruct((3, B, H), jnp.int32),
    )(x)


def kernel(x, embed, embed_bigram, embed_trigram, fc_w, fc_b):
    tbl = _tc_compact(embed).reshape(2 * RP, D)
    pooled = _sc_pool(_tc_remap_idx(x), tbl)
    return _tc_logit(pooled, fc_w, fc_b.reshape(1, C))


# speedup vs baseline: 1.7597x; 1.1156x over previous
import functools

import jax
import jax.numpy as jnp
from jax import lax
from jax.experimental import pallas as pl
from jax.experimental.pallas import tpu as pltpu
from jax.experimental.pallas import tpu_sc as plsc

V = 1000000
D = 64
C = 128
B = 4096
H = 200
G = 3 * H
NC, NS, L = 2, 16, 16
NW = NC * NS
BPW = B // NW
GRP = 16
NGRP = BPW // GRP
CHUNKS = ((0, 104), (104, 96))


def _sc_pool(x, embed):
    mesh = plsc.VectorSubcoreMesh(core_axis_name="c", subcore_axis_name="s")

    @functools.partial(
        pl.kernel,
        mesh=mesh,
        compiler_params=pltpu.CompilerParams(use_tc_tiling_on_sc=False),
        out_type=jax.ShapeDtypeStruct((B, D), jnp.float32),
        scratch_types=[
            pltpu.VMEM((2, 3, GRP, H), jnp.int32),
            pltpu.VMEM((2, G, D), jnp.float32),
            pltpu.VMEM((BPW, D), jnp.float32),
            pltpu.SemaphoreType.DMA,
            pltpu.SemaphoreType.DMA,
            pltpu.SemaphoreType.DMA,
        ],
    )
    def k(x_hbm, emb_hbm, out_hbm, idx_v, rows_v, acc_v, sem_i, sem_r0, sem_r1):
        wid = lax.axis_index("s") * NC + lax.axis_index("c")
        base = wid * BPW

        def issue_idx(grp, buf):
            for g in range(3):
                pltpu.async_copy(
                    x_hbm.at[g, pl.ds(base + grp * GRP, GRP), :],
                    idx_v.at[buf, g],
                    sem_i,
                )

        def wait_idx():
            for g in range(3):
                pltpu.make_async_copy(
                    x_hbm.at[g, pl.ds(base, GRP), :], idx_v.at[0, g], sem_i
                ).wait()

        def issue_gathers(buf, r, rbuf):
            sem = sem_r0 if rbuf == 0 else sem_r1
            for g in range(3):
                for off, sz in CHUNKS:
                    pltpu.async_copy(
                        emb_hbm.at[idx_v.at[buf, g, r, pl.ds(off, sz)]],
                        rows_v.at[rbuf, pl.ds(g * H + off, sz)],
                        sem,
                    )

        def wait_gathers(rbuf):
            sem = sem_r0 if rbuf == 0 else sem_r1
            for g in range(3):
                for off, sz in CHUNKS:
                    pltpu.make_async_copy(
                        emb_hbm.at[pl.ds(0, sz)],
                        rows_v.at[rbuf, pl.ds(g * H + off, sz)],
                        sem,
                    ).wait()

        def accumulate(rbuf, out_row):
            def body(j, accs):
                return tuple(
                    a + rows_v[rbuf, j, pl.ds(k2 * L, L)]
                    for k2, a in enumerate(accs)
                )

            accs = lax.fori_loop(
                0, G, body,
                tuple(jnp.zeros((L,), jnp.float32) for _ in range(D // L)),
                unroll=8,
            )
            for k2, a in enumerate(accs):
                acc_v[out_row, pl.ds(k2 * L, L)] = a

        issue_idx(0, 0)
        wait_idx()
        issue_idx(1, 1)
        issue_gathers(0, 0, 0)

        def grp_body(grp, carry):
            buf = lax.rem(grp, 2)
            nbuf = lax.rem(grp + 1, 2)
            for r in range(GRP):
                rbuf = r % 2
                if r < GRP - 1:
                    issue_gathers(buf, r + 1, (r + 1) % 2)
                else:
                    wait_idx()

                    @pl.when(grp + 2 < NGRP)
                    def _():
                        issue_idx(grp + 2, buf)

                    issue_gathers(nbuf, 0, 0)
                wait_gathers(rbuf)
                accumulate(rbuf, grp * GRP + r)
            return carry

        lax.fori_loop(0, NGRP - 1, grp_body, 0)

        fbuf = (NGRP - 1) % 2
        for r in range(GRP):
            rbuf = r % 2
            if r < GRP - 1:
                issue_gathers(fbuf, r + 1, (r + 1) % 2)
            wait_gathers(rbuf)
            accumulate(rbuf, (NGRP - 1) * GRP + r)

        pltpu.sync_copy(acc_v, out_hbm.at[pl.ds(base, BPW)])

    return k(x, embed)


def _tc_logit(pooled, fc_w, fc_b2):
    BLK = 512

    def body(p_ref, w_ref, b_ref, o_ref):
        p = p_ref[...] * (1.0 / H)
        o_ref[...] = (
            lax.dot_general(
                p, w_ref[...], (((1,), (1,)), ((), ())),
                preferred_element_type=jnp.float32,
            )
            + b_ref[...]
        )

    return pl.pallas_call(
        body,
        grid=(B // BLK,),
        in_specs=[
            pl.BlockSpec((BLK, D), lambda i: (i, 0)),
            pl.BlockSpec((C, D), lambda i: (0, 0)),
            pl.BlockSpec((1, C), lambda i: (0, 0)),
        ],
        out_specs=pl.BlockSpec((BLK, C), lambda i: (i, 0)),
        out_shape=jax.ShapeDtypeStruct((B, C), jnp.float32),
    )(pooled, fc_w, fc_b2)


CB = 4096
NB = (V + 2 * CB - 1) // (2 * CB)
RP = NB * CB


def _tc_compact(embed):
    embT = embed.T

    def body(a_ref, b_ref, o_ref):
        eye = jnp.eye(D, dtype=jnp.float32)
        t = lambda m: lax.dot_general(
            m, eye, (((0,), (0,)), ((), ())), preferred_element_type=jnp.float32
        )
        o_ref[...] = jnp.concatenate([t(a_ref[...]), t(b_ref[...])], axis=1)

    return pl.pallas_call(
        body,
        grid=(NB,),
        in_specs=[
            pl.BlockSpec((D, CB), lambda b: (0, 2 * b)),
            pl.BlockSpec((D, CB), lambda b: (0, jnp.minimum(2 * b + 1, V // CB))),
        ],
        out_specs=pl.BlockSpec((CB, 2 * D), lambda b: (b, 0)),
        out_shape=jax.ShapeDtypeStruct((RP, 2 * D), jnp.float32),
    )(embT, embT)


def _tc_remap_idx(x):
    XB = 512

    def body(x_ref, o_ref):
        v = x_ref[...]
        j = jnp.bitwise_and(v, 2 * CB - 1)
        o_ref[...] = v + j - jnp.where(j >= CB, 2 * CB - 1, 0)

    return pl.pallas_call(
        body,
        grid=(B // XB,),
        in_specs=[pl.BlockSpec((3, XB, H), lambda i: (0, i, 0))],
        out_specs=pl.BlockSpec((3, XB, H), lambda i: (0, i, 0)),
        out_shape=jax.ShapeDtypeStruct((3, B, H), jnp.int32),
    )(x)


def kernel(x, embed, embed_bigram, embed_trigram, fc_w, fc_b):
    tbl = _tc_compact(embed).reshape(2 * RP, D)
    pooled = _sc_pool(_tc_remap_idx(x), tbl)
    return _tc_logit(pooled, fc_w, fc_b.reshape(1, C))
